# baseline (speedup 1.0000x reference)
import functools

import jax
import jax.numpy as jnp
from jax import lax
from jax.experimental import pallas as pl
from jax.experimental.pallas import tpu as pltpu
from jax.experimental.pallas import tpu_sc as plsc

SEX_DIM = 4
SITE_DIM = 100000
EMBED_DIM = 64
BATCH = 16384

_info = plsc.get_sparse_core_info()
NC, NS, L = _info.num_cores, _info.num_subcores, _info.num_lanes
NW = NC * NS
B_PER_W = BATCH // NW
CHUNK = 128
HALF = 256
GROUPS = HALF // L

_mesh = plsc.VectorSubcoreMesh(core_axis_name="c", subcore_axis_name="s")


@functools.partial(
    pl.kernel,
    out_type=jax.ShapeDtypeStruct((BATCH, 2 * EMBED_DIM), jnp.float32),
    mesh=_mesh,
    compiler_params=pltpu.CompilerParams(needs_layout_passes=False),
    scratch_types=[
        pltpu.VMEM((B_PER_W,), jnp.int32),
        pltpu.VMEM((B_PER_W,), jnp.int32),
        pltpu.VMEM((B_PER_W,), jnp.int32),
        pltpu.VMEM((SEX_DIM, EMBED_DIM), jnp.float32),
        pltpu.VMEM((HALF, 2 * EMBED_DIM), jnp.float32),
        pltpu.VMEM((HALF, 2 * EMBED_DIM), jnp.float32),
        pltpu.SemaphoreType.DMA,
    ],
)
def _encoder_kernel(sex_hbm, site_hbm, sex_table_hbm, site_pair_hbm,
                    out_hbm, sexidx_v, siteidx_v, pairidx_v, sextab_v,
                    pairbuf_v, cat_v, sem):
    wid = lax.axis_index("s") * NC + lax.axis_index("c")
    base = wid * B_PER_W
    lane = lax.iota(jnp.int32, L)

    pltpu.sync_copy(sex_hbm.at[wid], sexidx_v)
    pltpu.sync_copy(site_hbm.at[wid], siteidx_v)
    pltpu.sync_copy(sex_table_hbm, sextab_v)

    for c in range(B_PER_W // L):
        pairidx_v[pl.ds(c * L, L)] = siteidx_v[pl.ds(c * L, L)] >> 1

    for h in range(B_PER_W // HALF):
        gathers = []
        for c in range(HALF // CHUNK):
            gathers.append(pltpu.async_copy(
                site_pair_hbm.at[
                    pairidx_v.at[pl.ds(h * HALF + c * CHUNK, CHUNK)]],
                pairbuf_v.at[pl.ds(c * CHUNK, CHUNK)], sem))
        for g in gathers:
            g.wait()

        def body(g, carry):
            k0 = g * L
            s_vec = siteidx_v[pl.ds(h * HALF + k0, L)]
            a_vec = sexidx_v[pl.ds(h * HALF + k0, L)]
            rowv = k0 + lane
            sitecol = (s_vec & 1) << 6
            for j in range(EMBED_DIM):
                jv = jnp.full((L,), j, jnp.int32)
                v = plsc.load_gather(pairbuf_v, [rowv, sitecol + j])
                plsc.store_scatter(cat_v, [rowv, jv + EMBED_DIM], v)
                w = plsc.load_gather(sextab_v, [a_vec, jv])
                plsc.store_scatter(cat_v, [rowv, jv], w)
            return carry

        lax.fori_loop(0, GROUPS, body, 0)
        pltpu.sync_copy(cat_v, out_hbm.at[pl.ds(base + h * HALF, HALF)])


@jax.jit
def kernel(sex, site, sex_table, site_table):
    sex_i = sex.astype(jnp.int32).reshape(NW, B_PER_W)
    site_i = site.astype(jnp.int32).reshape(NW, B_PER_W)
    site_pair = site_table.reshape(SITE_DIM // 2, 2 * EMBED_DIM)
    return _encoder_kernel(sex_i, site_i, sex_table, site_pair)


# speedup vs baseline: 1.5752x; 1.5752x over previous
import functools

import jax
import jax.numpy as jnp
from jax import lax
from jax.experimental import pallas as pl
from jax.experimental.pallas import tpu as pltpu
from jax.experimental.pallas import tpu_sc as plsc

SEX_DIM = 4
SITE_DIM = 100000
EMBED_DIM = 64
BATCH = 16384

_info = plsc.get_sparse_core_info()
NC, NS, L = _info.num_cores, _info.num_subcores, _info.num_lanes
NW = NC * NS
B_PER_W = BATCH // NW
CHUNK = 128
HALF = 256
GROUPS = HALF // L

_mesh = plsc.VectorSubcoreMesh(core_axis_name="c", subcore_axis_name="s")


@functools.partial(
    pl.kernel,
    out_type=jax.ShapeDtypeStruct((BATCH, 2 * EMBED_DIM), jnp.float32),
    mesh=_mesh,
    compiler_params=pltpu.CompilerParams(needs_layout_passes=False),
    scratch_types=[
        pltpu.VMEM((B_PER_W,), jnp.int32),
        pltpu.VMEM((B_PER_W,), jnp.int32),
        pltpu.VMEM((B_PER_W,), jnp.int32),
        pltpu.VMEM((SEX_DIM, EMBED_DIM), jnp.float32),
        pltpu.VMEM((HALF, 2 * EMBED_DIM), jnp.float32),
        pltpu.VMEM((HALF, 2 * EMBED_DIM), jnp.float32),
        pltpu.SemaphoreType.DMA,
    ],
)
def _encoder_kernel(sex_hbm, site_hbm, sex_table_hbm, site_pair_hbm,
                    out_hbm, sexidx_v, siteidx_v, pairidx_v, sextab_v,
                    pairbuf_v, cat_v, sem):
    wid = lax.axis_index("s") * NC + lax.axis_index("c")
    base = wid * B_PER_W
    lane = lax.iota(jnp.int32, L)

    pltpu.sync_copy(sex_hbm.at[wid], sexidx_v)
    pltpu.sync_copy(site_hbm.at[wid], siteidx_v)
    pltpu.sync_copy(sex_table_hbm, sextab_v)

    for c in range(B_PER_W // L):
        pairidx_v[pl.ds(c * L, L)] = siteidx_v[pl.ds(c * L, L)] >> 1

    for h in range(B_PER_W // HALF):
        gathers = []
        for c in range(HALF // CHUNK):
            gathers.append(pltpu.async_copy(
                site_pair_hbm.at[
                    pairidx_v.at[pl.ds(h * HALF + c * CHUNK, CHUNK)]],
                pairbuf_v.at[pl.ds(c * CHUNK, CHUNK)], sem))
        for g in gathers:
            g.wait()

        def body(g, carry):
            k0 = g * L
            s_vec = siteidx_v[pl.ds(h * HALF + k0, L)]
            a_vec = sexidx_v[pl.ds(h * HALF + k0, L)]
            r_vec = (s_vec & 1) * EMBED_DIM
            for i in range(L):
                r = r_vec[i]
                a = a_vec[i]
                k = k0 + i
                for jj in range(EMBED_DIM // L):
                    cat_v[k, pl.ds(EMBED_DIM + jj * L, L)] = (
                        pairbuf_v[k, pl.ds(r + jj * L, L)])
                    cat_v[k, pl.ds(jj * L, L)] = (
                        sextab_v[a, pl.ds(jj * L, L)])
            return carry

        lax.fori_loop(0, GROUPS, body, 0)
        pltpu.sync_copy(cat_v, out_hbm.at[pl.ds(base + h * HALF, HALF)])


@jax.jit
def kernel(sex, site, sex_table, site_table):
    sex_i = sex.astype(jnp.int32).reshape(NW, B_PER_W)
    site_i = site.astype(jnp.int32).reshape(NW, B_PER_W)
    site_pair = site_table.reshape(SITE_DIM // 2, 2 * EMBED_DIM)
    return _encoder_kernel(sex_i, site_i, sex_table, site_pair)


# speedup vs baseline: 2.1164x; 1.3436x over previous
import functools

import jax
import jax.numpy as jnp
from jax import lax
from jax.experimental import pallas as pl
from jax.experimental.pallas import tpu as pltpu
from jax.experimental.pallas import tpu_sc as plsc

SEX_DIM = 4
SITE_DIM = 100000
EMBED_DIM = 64
BATCH = 16384

_info = plsc.get_sparse_core_info()
NC, NS, L = _info.num_cores, _info.num_subcores, _info.num_lanes
NW = NC * NS
B_PER_W = BATCH // NW
GROUPS = B_PER_W // L

_mesh = plsc.VectorSubcoreMesh(core_axis_name="c", subcore_axis_name="s")


@functools.partial(
    pl.kernel,
    out_type=jax.ShapeDtypeStruct((BATCH, 2 * EMBED_DIM), jnp.float32),
    mesh=_mesh,
    compiler_params=pltpu.CompilerParams(needs_layout_passes=False),
    scratch_types=[
        pltpu.VMEM((B_PER_W,), jnp.int32),
        pltpu.VMEM((B_PER_W,), jnp.int32),
        pltpu.VMEM((SEX_DIM, EMBED_DIM), jnp.float32),
        pltpu.VMEM((B_PER_W, 2 * EMBED_DIM), jnp.float32),
        pltpu.SemaphoreType.DMA,
    ],
)
def _encoder_kernel(sex_hbm, site_hbm, sex_table_hbm, site_table_hbm,
                    out_hbm, sexidx_v, siteidx_v, sextab_v, cat_v, sem):
    wid = lax.axis_index("s") * NC + lax.axis_index("c")
    base = wid * B_PER_W

    pltpu.sync_copy(sex_hbm.at[wid], sexidx_v)
    pltpu.sync_copy(site_hbm.at[wid], siteidx_v)
    pltpu.sync_copy(sex_table_hbm, sextab_v)

    GROUPS_PER_WAVE = 8
    NWAVES = GROUPS // GROUPS_PER_WAVE
    waves = []
    for w in range(NWAVES):
        descs = []
        for g in range(w * GROUPS_PER_WAVE, (w + 1) * GROUPS_PER_WAVE):
            k0 = g * L
            s_vec = siteidx_v[pl.ds(k0, L)]
            for i in range(L):
                descs.append(pltpu.async_copy(
                    site_table_hbm.at[s_vec[i]],
                    cat_v.at[k0 + i, pl.ds(EMBED_DIM, EMBED_DIM)], sem))
        waves.append(descs)
        if w >= 2:
            for d in waves[w - 2]:
                d.wait()
    def body(g, carry):
        k0 = g * L
        a_vec = sexidx_v[pl.ds(k0, L)]
        for i in range(L):
            a = a_vec[i]
            for jj in range(EMBED_DIM // L):
                cat_v[k0 + i, pl.ds(jj * L, L)] = (
                    sextab_v[a, pl.ds(jj * L, L)])
        return carry

    lax.fori_loop(0, GROUPS, body, 0)

    for descs in waves[max(0, NWAVES - 2):]:
        for d in descs:
            d.wait()

    pltpu.sync_copy(cat_v, out_hbm.at[pl.ds(base, B_PER_W)])


@jax.jit
def kernel(sex, site, sex_table, site_table):
    sex_i = sex.astype(jnp.int32).reshape(NW, B_PER_W)
    site_i = site.astype(jnp.int32).reshape(NW, B_PER_W)
    return _encoder_kernel(sex_i, site_i, sex_table, site_table)


# speedup vs baseline: 2.1309x; 1.0068x over previous
import functools

import jax
import jax.numpy as jnp
from jax import lax
from jax.experimental import pallas as pl
from jax.experimental.pallas import tpu as pltpu
from jax.experimental.pallas import tpu_sc as plsc

SEX_DIM = 4
SITE_DIM = 100000
EMBED_DIM = 64
BATCH = 16384

_info = plsc.get_sparse_core_info()
NC, NS, L = _info.num_cores, _info.num_subcores, _info.num_lanes
NW = NC * NS
B_PER_W = BATCH // NW
GROUPS = B_PER_W // L

_mesh = plsc.VectorSubcoreMesh(core_axis_name="c", subcore_axis_name="s")


@functools.partial(
    pl.kernel,
    out_type=jax.ShapeDtypeStruct((BATCH, 2 * EMBED_DIM), jnp.float32),
    mesh=_mesh,
    compiler_params=pltpu.CompilerParams(
        needs_layout_passes=False, skip_device_barrier=True),
    scratch_types=[
        pltpu.VMEM((B_PER_W,), jnp.int32),
        pltpu.VMEM((B_PER_W,), jnp.int32),
        pltpu.VMEM((SEX_DIM, EMBED_DIM), jnp.float32),
        pltpu.VMEM((B_PER_W, 2 * EMBED_DIM), jnp.float32),
        pltpu.SemaphoreType.DMA,
    ],
)
def _encoder_kernel(sex_hbm, site_hbm, sex_table_hbm, site_table_hbm,
                    out_hbm, sexidx_v, siteidx_v, sextab_v, cat_v, sem):
    wid = lax.axis_index("s") * NC + lax.axis_index("c")
    base = wid * B_PER_W

    pltpu.sync_copy(sex_hbm.at[wid], sexidx_v)
    pltpu.sync_copy(site_hbm.at[wid], siteidx_v)
    pltpu.sync_copy(sex_table_hbm, sextab_v)

    GROUPS_PER_WAVE = 8
    NWAVES = GROUPS // GROUPS_PER_WAVE
    waves = []
    for w in range(NWAVES):
        descs = []
        for g in range(w * GROUPS_PER_WAVE, (w + 1) * GROUPS_PER_WAVE):
            k0 = g * L
            s_vec = siteidx_v[pl.ds(k0, L)]
            for i in range(L):
                descs.append(pltpu.async_copy(
                    site_table_hbm.at[s_vec[i]],
                    cat_v.at[k0 + i, pl.ds(EMBED_DIM, EMBED_DIM)], sem))
        waves.append(descs)
        if w >= 2:
            for d in waves[w - 2]:
                d.wait()
    def body(g, carry):
        k0 = g * L
        a_vec = sexidx_v[pl.ds(k0, L)]
        for i in range(L):
            a = a_vec[i]
            for jj in range(EMBED_DIM // L):
                cat_v[k0 + i, pl.ds(jj * L, L)] = (
                    sextab_v[a, pl.ds(jj * L, L)])
        return carry

    lax.fori_loop(0, GROUPS, body, 0)

    for descs in waves[max(0, NWAVES - 2):]:
        for d in descs:
            d.wait()

    pltpu.sync_copy(cat_v, out_hbm.at[pl.ds(base, B_PER_W)])


@jax.jit
def kernel(sex, site, sex_table, site_table):
    sex_i = sex.astype(jnp.int32).reshape(NW, B_PER_W)
    site_i = site.astype(jnp.int32).reshape(NW, B_PER_W)
    return _encoder_kernel(sex_i, site_i, sex_table, site_table)


# speedup vs baseline: 2.1965x; 1.0308x over previous
import functools

import jax
import jax.numpy as jnp
from jax import lax
from jax.experimental import pallas as pl
from jax.experimental.pallas import tpu as pltpu
from jax.experimental.pallas import tpu_sc as plsc

SEX_DIM = 4
SITE_DIM = 100000
EMBED_DIM = 64
BATCH = 16384

_info = plsc.get_sparse_core_info()
NC, NS, L = _info.num_cores, _info.num_subcores, _info.num_lanes
NW = NC * NS
B_PER_W = BATCH // NW
GROUPS = B_PER_W // L

_mesh = plsc.VectorSubcoreMesh(core_axis_name="c", subcore_axis_name="s")


@functools.partial(
    pl.kernel,
    out_type=jax.ShapeDtypeStruct((BATCH, 2 * EMBED_DIM), jnp.float32),
    mesh=_mesh,
    compiler_params=pltpu.CompilerParams(needs_layout_passes=False),
    scratch_types=[
        pltpu.VMEM((B_PER_W,), jnp.int32),
        pltpu.VMEM((B_PER_W,), jnp.int32),
        pltpu.VMEM((SEX_DIM, EMBED_DIM), jnp.float32),
        pltpu.VMEM((B_PER_W, 2 * EMBED_DIM), jnp.float32),
        pltpu.SemaphoreType.DMA,
        pltpu.SemaphoreType.DMA,
    ],
)
def _encoder_kernel(sex_hbm, site_hbm, sex_table_hbm, site_table_hbm,
                    out_hbm, sexidx_v, siteidx_v, sextab_v, cat_v, sem,
                    osem):
    wid = lax.axis_index("s") * NC + lax.axis_index("c")
    base = wid * B_PER_W

    c1 = pltpu.async_copy(site_hbm.at[wid], siteidx_v, osem)
    c2 = pltpu.async_copy(sex_hbm.at[wid], sexidx_v, osem)
    c3 = pltpu.async_copy(sex_table_hbm, sextab_v, osem)
    c1.wait()

    GROUPS_PER_WAVE = 8
    NWAVES = GROUPS // GROUPS_PER_WAVE
    WROWS = GROUPS_PER_WAVE * L

    def issue_wave(w):
        descs = []
        for g in range(w * GROUPS_PER_WAVE, (w + 1) * GROUPS_PER_WAVE):
            k0 = g * L
            s_vec = siteidx_v[pl.ds(k0, L)]
            for i in range(L):
                descs.append(pltpu.async_copy(
                    site_table_hbm.at[s_vec[i]],
                    cat_v.at[k0 + i, pl.ds(EMBED_DIM, EMBED_DIM)], sem))
        return descs

    def sex_fill(w):
        def body(g, carry):
            k0 = g * L
            a_vec = sexidx_v[pl.ds(k0, L)]
            for i in range(L):
                a = a_vec[i]
                for jj in range(EMBED_DIM // L):
                    cat_v[k0 + i, pl.ds(jj * L, L)] = (
                        sextab_v[a, pl.ds(jj * L, L)])
            return carry
        lax.fori_loop(w * GROUPS_PER_WAVE, (w + 1) * GROUPS_PER_WAVE,
                      body, 0)

    def write_wave(w):
        return pltpu.async_copy(
            cat_v.at[pl.ds(w * WROWS, WROWS)],
            out_hbm.at[pl.ds(base + w * WROWS, WROWS)], osem)

    c2.wait()
    c3.wait()

    waves = [issue_wave(0)]
    out_descs = []
    for w in range(1, NWAVES):
        waves.append(issue_wave(w))
        sex_fill(w - 1)
        for d in waves[w - 1]:
            d.wait()
        out_descs.append(write_wave(w - 1))
    sex_fill(NWAVES - 1)
    for d in waves[NWAVES - 1]:
        d.wait()
    out_descs.append(write_wave(NWAVES - 1))
    for d in out_descs:
        d.wait()


@jax.jit
def kernel(sex, site, sex_table, site_table):
    sex_i = sex.astype(jnp.int32).reshape(NW, B_PER_W)
    site_i = site.astype(jnp.int32).reshape(NW, B_PER_W)
    return _encoder_kernel(sex_i, site_i, sex_table, site_table)


# speedup vs baseline: 2.2959x; 1.0452x over previous
import functools

import jax
import jax.numpy as jnp
from jax import lax
from jax.experimental import pallas as pl
from jax.experimental.pallas import tpu as pltpu
from jax.experimental.pallas import tpu_sc as plsc

SEX_DIM = 4
SITE_DIM = 100000
EMBED_DIM = 64
BATCH = 16384

_info = plsc.get_sparse_core_info()
NC, NS, L = _info.num_cores, _info.num_subcores, _info.num_lanes
NW = NC * NS
B_PER_W = BATCH // NW
GROUPS = B_PER_W // L

_mesh = plsc.VectorSubcoreMesh(core_axis_name="c", subcore_axis_name="s")


@functools.partial(
    pl.kernel,
    out_type=jax.ShapeDtypeStruct((BATCH, 2 * EMBED_DIM), jnp.float32),
    mesh=_mesh,
    compiler_params=pltpu.CompilerParams(needs_layout_passes=False),
    scratch_types=[
        pltpu.VMEM((B_PER_W,), jnp.int32),
        pltpu.VMEM((B_PER_W,), jnp.int32),
        pltpu.VMEM((SEX_DIM, EMBED_DIM), jnp.float32),
        pltpu.VMEM((B_PER_W, 2 * EMBED_DIM), jnp.float32),
        pltpu.SemaphoreType.DMA,
        pltpu.SemaphoreType.DMA,
    ],
)
def _encoder_kernel(sex_hbm, site_hbm, sex_table_hbm, site_table_hbm,
                    out_hbm, sexidx_v, siteidx_v, sextab_v, cat_v, sem,
                    osem):
    wid = lax.axis_index("s") * NC + lax.axis_index("c")
    base = wid * B_PER_W

    c1 = pltpu.async_copy(site_hbm.at[pl.ds(base, B_PER_W)], siteidx_v, osem)
    c2 = pltpu.async_copy(sex_hbm.at[pl.ds(base, B_PER_W)], sexidx_v, osem)
    c3 = pltpu.async_copy(sex_table_hbm, sextab_v, osem)
    c1.wait()

    GROUPS_PER_WAVE = 8
    NWAVES = GROUPS // GROUPS_PER_WAVE
    WROWS = GROUPS_PER_WAVE * L

    def issue_wave(w):
        descs = []
        for g in range(w * GROUPS_PER_WAVE, (w + 1) * GROUPS_PER_WAVE):
            k0 = g * L
            s_vec = siteidx_v[pl.ds(k0, L)]
            for i in range(L):
                descs.append(pltpu.async_copy(
                    site_table_hbm.at[s_vec[i]],
                    cat_v.at[k0 + i, pl.ds(EMBED_DIM, EMBED_DIM)], sem))
        return descs

    def sex_fill(w):
        def body(g, carry):
            k0 = g * L
            a_vec = sexidx_v[pl.ds(k0, L)]
            for i in range(L):
                a = a_vec[i]
                for jj in range(EMBED_DIM // L):
                    cat_v[k0 + i, pl.ds(jj * L, L)] = (
                        sextab_v[a, pl.ds(jj * L, L)])
            return carry
        lax.fori_loop(w * GROUPS_PER_WAVE, (w + 1) * GROUPS_PER_WAVE,
                      body, 0)

    def write_wave(w):
        return pltpu.async_copy(
            cat_v.at[pl.ds(w * WROWS, WROWS)],
            out_hbm.at[pl.ds(base + w * WROWS, WROWS)], osem)

    c2.wait()
    c3.wait()

    waves = [issue_wave(0)]
    out_descs = []
    for w in range(1, NWAVES):
        waves.append(issue_wave(w))
        sex_fill(w - 1)
        for d in waves[w - 1]:
            d.wait()
        out_descs.append(write_wave(w - 1))
    sex_fill(NWAVES - 1)
    for d in waves[NWAVES - 1]:
        d.wait()
    out_descs.append(write_wave(NWAVES - 1))
    for d in out_descs:
        d.wait()


@jax.jit
def kernel(sex, site, sex_table, site_table):
    return _encoder_kernel(sex.astype(jnp.int32), site.astype(jnp.int32),
                           sex_table, site_table)
